# baseline (speedup 1.0000x reference)
import functools

import jax
import jax.numpy as jnp
from jax import lax
from jax.experimental import pallas as pl
from jax.experimental.pallas import tpu as pltpu
from jax.experimental.pallas import tpu_sc as plsc

VOCAB = 1000000
D = 64
NCLS = 100
B = 16384
H = 50

HALF = 500224

NC, NS = 2, 16
NW = NC * NS
SPW = B // NW
CH = 16
NCHUNK = SPW // CH
RPC = CH * H


def _sc_pool_sums(table, xf):
  mesh = plsc.VectorSubcoreMesh(
      core_axis_name="c", subcore_axis_name="s", num_cores=NC, num_subcores=NS
  )

  @functools.partial(
      pl.kernel,
      out_type=jax.ShapeDtypeStruct((B, D), jnp.float32),
      mesh=mesh,
      scratch_types=[
          pltpu.VMEM((2, CH, H), jnp.int32),
          pltpu.VMEM((2, RPC, D), jnp.float32),
          pltpu.VMEM((CH, D), jnp.float32),
          pltpu.SemaphoreType.DMA,
          pltpu.SemaphoreType.DMA,
      ],
      compiler_params=pltpu.CompilerParams(use_tc_tiling_on_sc=False),
  )
  def k(table_hbm, xf_hbm, out_hbm, idx_v, rows_v, pool_v, sem0, sem1):
    wid = lax.axis_index("s") * NC + lax.axis_index("c")
    sems = (sem0, sem1)

    def load(cc, slot):
      sbase = wid * SPW + cc * CH
      pltpu.sync_copy(xf_hbm.at[pl.ds(sbase, CH)], idx_v.at[slot])
      for j in range(CH):
        pltpu.async_copy(
            table_hbm.at[idx_v.at[slot].at[j]],
            rows_v.at[slot].at[pl.ds(j * H, H)],
            sems[slot],
        )

    def drain(slot):
      pltpu.make_async_copy(
          table_hbm.at[pl.ds(0, RPC)], rows_v.at[slot], sems[slot]
      ).wait()

    def pool_store(cc, slot):
      @pl.loop(0, CH)
      def _sample(s):
        base = s * H

        def body(l, accs):
          r = base + l * 5
          out = accs
          for u in range(5):
            out = tuple(
                out[v] + rows_v[slot, r + u, pl.ds(v * 16, 16)]
                for v in range(4)
            )
          return out

        accs = lax.fori_loop(
            0, H // 5, body,
            tuple(jnp.zeros((16,), jnp.float32) for _ in range(4)),
        )
        for v in range(4):
          pool_v[s, pl.ds(v * 16, 16)] = accs[v]

      sbase = wid * SPW + cc * CH
      pltpu.sync_copy(pool_v, out_hbm.at[pl.ds(sbase, CH)])

    load(0, 0)

    @pl.loop(0, NCHUNK, step=2)
    def _chunk(c):
      for b in range(2):
        cc = c + b

        @pl.when(cc + 1 < NCHUNK)
        def _():
          load(cc + 1, (b + 1) % 2)

        drain(b)
        pool_store(cc, b)

  return k(table, xf)


def _tc_transpose(tt):
  bn = 512
  nb = HALF // bn

  def body(a_ref, b_ref, o_ref):
    o_ref[:, 0:D] = a_ref[...].T
    o_ref[:, D:128] = b_ref[...].T

  return pl.pallas_call(
      body,
      grid=(nb,),
      in_specs=[
          pl.BlockSpec((D, bn), lambda i: (0, i)),
          pl.BlockSpec((D, bn), lambda i: (0, i + nb)),
      ],
      out_specs=pl.BlockSpec((bn, 128), lambda i: (i, 0)),
      out_shape=jax.ShapeDtypeStruct((HALF, 128), jnp.float32),
  )(tt, tt)


def _tc_linear(pooled, wt_pad, b_pad):
  bm = 2048

  def body(p_ref, wt_ref, b_ref, o_ref):
    acc = jnp.dot(p_ref[...], wt_ref[...], preferred_element_type=jnp.float32)
    o_ref[...] = acc * (1.0 / H) + b_ref[...]

  return pl.pallas_call(
      body,
      grid=(B // bm,),
      in_specs=[
          pl.BlockSpec((bm, D), lambda i: (i, 0)),
          pl.BlockSpec((D, 128), lambda i: (0, 0)),
          pl.BlockSpec((1, 128), lambda i: (0, 0)),
      ],
      out_specs=pl.BlockSpec((bm, 128), lambda i: (i, 0)),
      out_shape=jax.ShapeDtypeStruct((B, 128), jnp.float32),
  )(pooled, wt_pad, b_pad)


def kernel(x, table, W, b):
  xi = x.astype(jnp.int32)
  tab = _tc_transpose(table.T).reshape(2 * HALF, D)
  xf = (xi % HALF) * 2 + xi // HALF
  pooled = _sc_pool_sums(tab, xf)
  wt_pad = jnp.zeros((D, 128), jnp.float32).at[:, :NCLS].set(W.T)
  b_pad = jnp.zeros((1, 128), jnp.float32).at[:, :NCLS].set(b.reshape(1, -1))
  out = _tc_linear(pooled, wt_pad, b_pad)
  return out[:, :NCLS]


# speedup vs baseline: 2.0182x; 2.0182x over previous
import functools

import jax
import jax.numpy as jnp
from jax import lax
from jax.experimental import pallas as pl
from jax.experimental.pallas import tpu as pltpu
from jax.experimental.pallas import tpu_sc as plsc

VOCAB = 1000000
D = 64
NCLS = 100
B = 16384
H = 50

TBN = 4096
HALF = 503808

NC, NS = 2, 16
NW = NC * NS
SPW = B // NW
CH = 16
NCHUNK = SPW // CH
RPC = CH * H


def _sc_pool_sums(table, xf):
  mesh = plsc.VectorSubcoreMesh(
      core_axis_name="c", subcore_axis_name="s", num_cores=NC, num_subcores=NS
  )

  @functools.partial(
      pl.kernel,
      out_type=jax.ShapeDtypeStruct((B, D), jnp.float32),
      mesh=mesh,
      scratch_types=[
          pltpu.VMEM((2, CH, H), jnp.int32),
          pltpu.VMEM((2, RPC, D), jnp.float32),
          pltpu.VMEM((CH, D), jnp.float32),
          pltpu.SemaphoreType.DMA,
          pltpu.SemaphoreType.DMA,
      ],
      compiler_params=pltpu.CompilerParams(use_tc_tiling_on_sc=False),
  )
  def k(table_hbm, xf_hbm, out_hbm, idx_v, rows_v, pool_v, sem0, sem1):
    wid = lax.axis_index("s") * NC + lax.axis_index("c")
    sems = (sem0, sem1)

    def load(cc, slot):
      sbase = wid * SPW + cc * CH
      pltpu.sync_copy(xf_hbm.at[pl.ds(sbase, CH)], idx_v.at[slot])
      for j in range(CH):
        pltpu.async_copy(
            table_hbm.at[idx_v.at[slot].at[j]],
            rows_v.at[slot].at[pl.ds(j * H, H)],
            sems[slot],
        )

    def drain(slot):
      pltpu.make_async_copy(
          table_hbm.at[pl.ds(0, RPC)], rows_v.at[slot], sems[slot]
      ).wait()

    def pool_store(cc, slot):
      @pl.loop(0, CH)
      def _sample(s):
        base = s * H

        def body(l, accs):
          r = base + l * 5
          out = accs
          for u in range(5):
            out = tuple(
                out[v] + rows_v[slot, r + u, pl.ds(v * 16, 16)]
                for v in range(4)
            )
          return out

        accs = lax.fori_loop(
            0, H // 5, body,
            tuple(jnp.zeros((16,), jnp.float32) for _ in range(4)),
        )
        for v in range(4):
          pool_v[s, pl.ds(v * 16, 16)] = accs[v]

      sbase = wid * SPW + cc * CH
      pltpu.sync_copy(pool_v, out_hbm.at[pl.ds(sbase, CH)])

    load(0, 0)

    @pl.loop(0, NCHUNK, step=2)
    def _chunk(c):
      for b in range(2):
        cc = c + b

        @pl.when(cc + 1 < NCHUNK)
        def _():
          load(cc + 1, (b + 1) % 2)

        drain(b)
        pool_store(cc, b)

  return k(table, xf)


def _tc_transpose(tt):
  bn = TBN
  nb = HALF // bn
  nlast = (VOCAB - 1) // bn

  def body(a_ref, b_ref, o_ref):
    i64 = jnp.eye(D, dtype=jnp.float32)
    dn = (((0,), (0,)), ((), ()))
    o_ref[:, 0:D] = lax.dot_general(
        a_ref[...], i64, dn, preferred_element_type=jnp.float32
    )
    o_ref[:, D:128] = lax.dot_general(
        b_ref[...], i64, dn, preferred_element_type=jnp.float32
    )

  return pl.pallas_call(
      body,
      grid=(nb,),
      in_specs=[
          pl.BlockSpec((D, bn), lambda i: (0, i)),
          pl.BlockSpec((D, bn), lambda i: (0, jnp.minimum(i + nb, nlast))),
      ],
      out_specs=pl.BlockSpec((bn, 128), lambda i: (i, 0)),
      out_shape=jax.ShapeDtypeStruct((HALF, 128), jnp.float32),
  )(tt, tt)


def _tc_linear(pooled, wt_pad, b_pad):
  bm = 2048

  def body(p_ref, wt_ref, b_ref, o_ref):
    acc = jnp.dot(p_ref[...], wt_ref[...], preferred_element_type=jnp.float32)
    o_ref[...] = acc * (1.0 / H) + b_ref[...]

  return pl.pallas_call(
      body,
      grid=(B // bm,),
      in_specs=[
          pl.BlockSpec((bm, D), lambda i: (i, 0)),
          pl.BlockSpec((D, 128), lambda i: (0, 0)),
          pl.BlockSpec((1, 128), lambda i: (0, 0)),
      ],
      out_specs=pl.BlockSpec((bm, 128), lambda i: (i, 0)),
      out_shape=jax.ShapeDtypeStruct((B, 128), jnp.float32),
  )(pooled, wt_pad, b_pad)


def kernel(x, table, W, b):
  xi = x.astype(jnp.int32)
  tab = _tc_transpose(table.T).reshape(2 * HALF, D)
  xf = (xi % HALF) * 2 + xi // HALF
  pooled = _sc_pool_sums(tab, xf)
  wt_pad = jnp.zeros((D, 128), jnp.float32).at[:, :NCLS].set(W.T)
  b_pad = jnp.zeros((1, 128), jnp.float32).at[:, :NCLS].set(b.reshape(1, -1))
  out = _tc_linear(pooled, wt_pad, b_pad)
  return out[:, :NCLS]


# speedup vs baseline: 2.3774x; 1.1780x over previous
import functools

import jax
import jax.numpy as jnp
from jax import lax
from jax.experimental import pallas as pl
from jax.experimental.pallas import tpu as pltpu
from jax.experimental.pallas import tpu_sc as plsc

VOCAB = 1000000
D = 64
NCLS = 100
B = 16384
H = 50

TBN = 4096
HALF = 503808

NC, NS = 2, 16
NW = NC * NS
SPW = B // NW
CH = 16
NCHUNK = SPW // CH
RPC = CH * H


def _sc_pool_sums(table, xf):
  mesh = plsc.VectorSubcoreMesh(
      core_axis_name="c", subcore_axis_name="s", num_cores=NC, num_subcores=NS
  )

  @functools.partial(
      pl.kernel,
      out_type=jax.ShapeDtypeStruct((B, D), jnp.float32),
      mesh=mesh,
      scratch_types=[
          pltpu.VMEM((2, CH, H), jnp.int32),
          pltpu.VMEM((2, RPC, D), jnp.float32),
          pltpu.VMEM((CH, D), jnp.float32),
          pltpu.SemaphoreType.DMA,
          pltpu.SemaphoreType.DMA,
      ],
      compiler_params=pltpu.CompilerParams(use_tc_tiling_on_sc=False),
  )
  def k(table_hbm, xf_hbm, out_hbm, idx_v, rows_v, pool_v, sem0, sem1):
    wid = lax.axis_index("s") * NC + lax.axis_index("c")
    sems = (sem0, sem1)

    def load(cc, slot):
      sbase = wid * SPW + cc * CH
      pltpu.sync_copy(xf_hbm.at[pl.ds(sbase, CH)], idx_v.at[slot])
      for j in range(CH):
        pltpu.async_copy(
            table_hbm.at[idx_v.at[slot].at[j]],
            rows_v.at[slot].at[pl.ds(j * H, H)],
            sems[slot],
        )

    def drain(slot):
      pltpu.make_async_copy(
          table_hbm.at[pl.ds(0, RPC)], rows_v.at[slot], sems[slot]
      ).wait()

    def pool_store(cc, slot):
      @pl.loop(0, CH)
      def _sample(s):
        base = s * H

        def body(l, accs):
          r = base + l * 5
          out = accs
          for u in range(5):
            out = tuple(
                out[v] + rows_v[slot, r + u, pl.ds(v * 16, 16)]
                for v in range(4)
            )
          return out

        accs = lax.fori_loop(
            0, H // 5, body,
            tuple(jnp.zeros((16,), jnp.float32) for _ in range(4)),
        )
        for v in range(4):
          pool_v[s, pl.ds(v * 16, 16)] = accs[v]

      sbase = wid * SPW + cc * CH
      pltpu.sync_copy(pool_v, out_hbm.at[pl.ds(sbase, CH)])

    load(0, 0)

    @pl.loop(0, NCHUNK, step=2)
    def _chunk(c):
      for b in range(2):
        cc = c + b

        @pl.when(cc + 1 < NCHUNK)
        def _():
          load(cc + 1, (b + 1) % 2)

        drain(b)
        pool_store(cc, b)

  return k(table, xf)


def _tc_transpose(tt):
  bn = TBN
  nb = HALF // bn
  nlast = (VOCAB - 1) // bn

  def body(a_ref, b_ref, o_ref):
    i128 = jnp.eye(128, dtype=jnp.float32)
    dn = (((0,), (0,)), ((), ()))
    stacked = jnp.concatenate([a_ref[...], b_ref[...]], axis=0)
    o_ref[...] = lax.dot_general(
        stacked, i128, dn, preferred_element_type=jnp.float32
    )

  return pl.pallas_call(
      body,
      grid=(nb,),
      in_specs=[
          pl.BlockSpec((D, bn), lambda i: (0, i)),
          pl.BlockSpec((D, bn), lambda i: (0, jnp.minimum(i + nb, nlast))),
      ],
      out_specs=pl.BlockSpec((bn, 128), lambda i: (i, 0)),
      out_shape=jax.ShapeDtypeStruct((HALF, 128), jnp.float32),
  )(tt, tt)


def _tc_linear(pooled, wt_pad, b_pad):
  bm = 2048

  def body(p_ref, wt_ref, b_ref, o_ref):
    acc = jnp.dot(p_ref[...], wt_ref[...], preferred_element_type=jnp.float32)
    o_ref[...] = acc * (1.0 / H) + b_ref[...]

  return pl.pallas_call(
      body,
      grid=(B // bm,),
      in_specs=[
          pl.BlockSpec((bm, D), lambda i: (i, 0)),
          pl.BlockSpec((D, 128), lambda i: (0, 0)),
          pl.BlockSpec((1, 128), lambda i: (0, 0)),
      ],
      out_specs=pl.BlockSpec((bm, 128), lambda i: (i, 0)),
      out_shape=jax.ShapeDtypeStruct((B, 128), jnp.float32),
  )(pooled, wt_pad, b_pad)


def kernel(x, table, W, b):
  xi = x.astype(jnp.int32)
  tab = _tc_transpose(table.T).reshape(2 * HALF, D)
  xf = (xi % HALF) * 2 + xi // HALF
  pooled = _sc_pool_sums(tab, xf)
  wt_pad = jnp.zeros((D, 128), jnp.float32).at[:, :NCLS].set(W.T)
  b_pad = jnp.zeros((1, 128), jnp.float32).at[:, :NCLS].set(b.reshape(1, -1))
  out = _tc_linear(pooled, wt_pad, b_pad)
  return out[:, :NCLS]


# speedup vs baseline: 2.5748x; 1.0830x over previous
import functools

import jax
import jax.numpy as jnp
from jax import lax
from jax.experimental import pallas as pl
from jax.experimental.pallas import tpu as pltpu
from jax.experimental.pallas import tpu_sc as plsc

VOCAB = 1000000
D = 64
NCLS = 100
B = 16384
H = 50

TBN = 8192
HALF = 507904

NC, NS = 2, 16
NW = NC * NS
SPW = B // NW
CH = 16
NCHUNK = SPW // CH
RPC = CH * H


def _sc_pool_sums(table, xf):
  mesh = plsc.VectorSubcoreMesh(
      core_axis_name="c", subcore_axis_name="s", num_cores=NC, num_subcores=NS
  )

  @functools.partial(
      pl.kernel,
      out_type=jax.ShapeDtypeStruct((B, D), jnp.float32),
      mesh=mesh,
      scratch_types=[
          pltpu.VMEM((2, CH, H), jnp.int32),
          pltpu.VMEM((2, RPC, D), jnp.float32),
          pltpu.VMEM((CH, D), jnp.float32),
          pltpu.SemaphoreType.DMA,
          pltpu.SemaphoreType.DMA,
      ],
      compiler_params=pltpu.CompilerParams(use_tc_tiling_on_sc=False),
  )
  def k(table_hbm, xf_hbm, out_hbm, idx_v, rows_v, pool_v, sem0, sem1):
    wid = lax.axis_index("s") * NC + lax.axis_index("c")
    sems = (sem0, sem1)

    def load(cc, slot):
      sbase = wid * SPW + cc * CH
      pltpu.sync_copy(xf_hbm.at[pl.ds(sbase, CH)], idx_v.at[slot])
      for j in range(CH):
        pltpu.async_copy(
            table_hbm.at[idx_v.at[slot].at[j]],
            rows_v.at[slot].at[pl.ds(j * H, H)],
            sems[slot],
        )

    def drain(slot):
      pltpu.make_async_copy(
          table_hbm.at[pl.ds(0, RPC)], rows_v.at[slot], sems[slot]
      ).wait()

    def pool_store(cc, slot):
      @pl.loop(0, CH)
      def _sample(s):
        base = s * H

        def body(l, accs):
          r = base + l * 5
          out = accs
          for u in range(5):
            out = tuple(
                out[v] + rows_v[slot, r + u, pl.ds(v * 16, 16)]
                for v in range(4)
            )
          return out

        accs = lax.fori_loop(
            0, H // 5, body,
            tuple(jnp.zeros((16,), jnp.float32) for _ in range(4)),
        )
        for v in range(4):
          pool_v[s, pl.ds(v * 16, 16)] = accs[v]

      sbase = wid * SPW + cc * CH
      pltpu.sync_copy(pool_v, out_hbm.at[pl.ds(sbase, CH)])

    load(0, 0)

    @pl.loop(0, NCHUNK, step=2)
    def _chunk(c):
      for b in range(2):
        cc = c + b

        @pl.when(cc + 1 < NCHUNK)
        def _():
          load(cc + 1, (b + 1) % 2)

        drain(b)
        pool_store(cc, b)

  return k(table, xf)


def _tc_transpose(tt):
  bn = TBN
  nb = HALF // bn
  nlast = (VOCAB - 1) // bn

  def body(a_ref, b_ref, o_ref):
    i128 = jnp.eye(128, dtype=jnp.float32)
    dn = (((0,), (0,)), ((), ()))
    stacked = jnp.concatenate([a_ref[...], b_ref[...]], axis=0)
    o_ref[...] = lax.dot_general(
        stacked, i128, dn, preferred_element_type=jnp.float32
    )

  return pl.pallas_call(
      body,
      grid=(nb,),
      in_specs=[
          pl.BlockSpec((D, bn), lambda i: (0, i)),
          pl.BlockSpec((D, bn), lambda i: (0, jnp.minimum(i + nb, nlast))),
      ],
      out_specs=pl.BlockSpec((bn, 128), lambda i: (i, 0)),
      out_shape=jax.ShapeDtypeStruct((HALF, 128), jnp.float32),
  )(tt, tt)


def _tc_linear(pooled, wt_pad, b_pad):
  bm = 2048

  def body(p_ref, wt_ref, b_ref, o_ref):
    acc = jnp.dot(p_ref[...], wt_ref[...], preferred_element_type=jnp.float32)
    o_ref[...] = acc * (1.0 / H) + b_ref[...]

  return pl.pallas_call(
      body,
      grid=(B // bm,),
      in_specs=[
          pl.BlockSpec((bm, D), lambda i: (i, 0)),
          pl.BlockSpec((D, 128), lambda i: (0, 0)),
          pl.BlockSpec((1, 128), lambda i: (0, 0)),
      ],
      out_specs=pl.BlockSpec((bm, 128), lambda i: (i, 0)),
      out_shape=jax.ShapeDtypeStruct((B, 128), jnp.float32),
  )(pooled, wt_pad, b_pad)


def kernel(x, table, W, b):
  xi = x.astype(jnp.int32)
  tab = _tc_transpose(table.T).reshape(2 * HALF, D)
  xf = (xi % HALF) * 2 + xi // HALF
  pooled = _sc_pool_sums(tab, xf)
  wt_pad = jnp.zeros((D, 128), jnp.float32).at[:, :NCLS].set(W.T)
  b_pad = jnp.zeros((1, 128), jnp.float32).at[:, :NCLS].set(b.reshape(1, -1))
  out = _tc_linear(pooled, wt_pad, b_pad)
  return out[:, :NCLS]


# speedup vs baseline: 2.6339x; 1.0229x over previous
import functools

import jax
import jax.numpy as jnp
from jax import lax
from jax.experimental import pallas as pl
from jax.experimental.pallas import tpu as pltpu
from jax.experimental.pallas import tpu_sc as plsc

VOCAB = 1000000
D = 64
NCLS = 100
B = 16384
H = 50

TBN = 8192
HALF = 507904

NC, NS = 2, 16
NW = NC * NS
SPW = B // NW
CH = 16
NCHUNK = SPW // CH
RPC = CH * H


def _sc_pool_sums(table, xf):
  mesh = plsc.VectorSubcoreMesh(
      core_axis_name="c", subcore_axis_name="s", num_cores=NC, num_subcores=NS
  )

  @functools.partial(
      pl.kernel,
      out_type=jax.ShapeDtypeStruct((B, D), jnp.float32),
      mesh=mesh,
      scratch_types=[
          pltpu.VMEM((2, CH // 2, 2 * H), jnp.int32),
          pltpu.VMEM((2, RPC, D), jnp.float32),
          pltpu.VMEM((CH, D), jnp.float32),
          pltpu.SemaphoreType.DMA,
          pltpu.SemaphoreType.DMA,
      ],
      compiler_params=pltpu.CompilerParams(use_tc_tiling_on_sc=False),
  )
  def k(table_hbm, xf_hbm, out_hbm, idx_v, rows_v, pool_v, sem0, sem1):
    wid = lax.axis_index("s") * NC + lax.axis_index("c")
    sems = (sem0, sem1)

    def load(cc, slot):
      pbase = (wid * SPW + cc * CH) // 2
      pltpu.sync_copy(xf_hbm.at[pl.ds(pbase, CH // 2)], idx_v.at[slot])
      for j in range(CH // 2):
        pltpu.async_copy(
            table_hbm.at[idx_v.at[slot].at[j]],
            rows_v.at[slot].at[pl.ds(j * 2 * H, 2 * H)],
            sems[slot],
        )

    def drain(slot):
      pltpu.make_async_copy(
          table_hbm.at[pl.ds(0, RPC)], rows_v.at[slot], sems[slot]
      ).wait()

    def pool_store(cc, slot):
      @pl.loop(0, CH)
      def _sample(s):
        base = s * H

        def body(l, accs):
          r = base + l * 5
          out = accs
          for u in range(5):
            out = tuple(
                out[v] + rows_v[slot, r + u, pl.ds(v * 16, 16)]
                for v in range(4)
            )
          return out

        accs = lax.fori_loop(
            0, H // 5, body,
            tuple(jnp.zeros((16,), jnp.float32) for _ in range(4)),
        )
        for v in range(4):
          pool_v[s, pl.ds(v * 16, 16)] = accs[v]

      sbase = wid * SPW + cc * CH
      pltpu.sync_copy(pool_v, out_hbm.at[pl.ds(sbase, CH)])

    load(0, 0)

    @pl.loop(0, NCHUNK, step=2)
    def _chunk(c):
      for b in range(2):
        cc = c + b

        @pl.when(cc + 1 < NCHUNK)
        def _():
          load(cc + 1, (b + 1) % 2)

        drain(b)
        pool_store(cc, b)

  return k(table, xf)


def _tc_transpose(tt):
  bn = TBN
  nb = HALF // bn
  nlast = (VOCAB - 1) // bn

  def body(a_ref, b_ref, o_ref):
    i128 = jnp.eye(128, dtype=jnp.float32)
    dn = (((0,), (0,)), ((), ()))
    stacked = jnp.concatenate([a_ref[...], b_ref[...]], axis=0)
    o_ref[...] = lax.dot_general(
        stacked, i128, dn, preferred_element_type=jnp.float32
    )

  return pl.pallas_call(
      body,
      grid=(nb,),
      in_specs=[
          pl.BlockSpec((D, bn), lambda i: (0, i)),
          pl.BlockSpec((D, bn), lambda i: (0, jnp.minimum(i + nb, nlast))),
      ],
      out_specs=pl.BlockSpec((bn, 128), lambda i: (i, 0)),
      out_shape=jax.ShapeDtypeStruct((HALF, 128), jnp.float32),
  )(tt, tt)


def _tc_linear(pooled, wt_pad, b_pad):
  bm = 2048

  def body(p_ref, wt_ref, b_ref, o_ref):
    acc = jnp.dot(p_ref[...], wt_ref[...], preferred_element_type=jnp.float32)
    o_ref[...] = acc * (1.0 / H) + b_ref[...]

  return pl.pallas_call(
      body,
      grid=(B // bm,),
      in_specs=[
          pl.BlockSpec((bm, D), lambda i: (i, 0)),
          pl.BlockSpec((D, 128), lambda i: (0, 0)),
          pl.BlockSpec((1, 128), lambda i: (0, 0)),
      ],
      out_specs=pl.BlockSpec((bm, 128), lambda i: (i, 0)),
      out_shape=jax.ShapeDtypeStruct((B, 128), jnp.float32),
  )(pooled, wt_pad, b_pad)


def kernel(x, table, W, b):
  xi = x.astype(jnp.int32)
  tab = _tc_transpose(table.T).reshape(2 * HALF, D)
  xf = ((xi % HALF) * 2 + xi // HALF).reshape(B * H // (2 * H), 2 * H)
  pooled = _sc_pool_sums(tab, xf)
  wt_pad = jnp.zeros((D, 128), jnp.float32).at[:, :NCLS].set(W.T)
  b_pad = jnp.zeros((1, 128), jnp.float32).at[:, :NCLS].set(b.reshape(1, -1))
  out = _tc_linear(pooled, wt_pad, b_pad)
  return out[:, :NCLS]


# speedup vs baseline: 2.6387x; 1.0018x over previous
import functools

import jax
import jax.numpy as jnp
from jax import lax
from jax.experimental import pallas as pl
from jax.experimental.pallas import tpu as pltpu
from jax.experimental.pallas import tpu_sc as plsc

VOCAB = 1000000
D = 64
NCLS = 100
B = 16384
H = 50

TBN = 8192
HALF = 507904

NC, NS = 2, 16
NW = NC * NS
SPW = B // NW
CH = 16
NCHUNK = SPW // CH
RPC = CH * H


def _sc_pool_sums(table, xf):
  mesh = plsc.VectorSubcoreMesh(
      core_axis_name="c", subcore_axis_name="s", num_cores=NC, num_subcores=NS
  )

  @functools.partial(
      pl.kernel,
      out_type=jax.ShapeDtypeStruct((B, D), jnp.float32),
      mesh=mesh,
      scratch_types=[
          pltpu.VMEM((2, RPC), jnp.int32),
          pltpu.VMEM((2, RPC, D), jnp.float32),
          pltpu.VMEM((CH, D), jnp.float32),
          pltpu.SemaphoreType.DMA,
          pltpu.SemaphoreType.DMA,
      ],
      compiler_params=pltpu.CompilerParams(
          use_tc_tiling_on_sc=False, needs_layout_passes=False
      ),
  )
  def k(table_hbm, xf_hbm, out_hbm, idx_v, rows_v, pool_v, sem0, sem1):
    wid = lax.axis_index("s") * NC + lax.axis_index("c")
    sems = (sem0, sem1)

    def load(cc, slot):
      rbase = (wid * SPW + cc * CH) * H
      pltpu.sync_copy(xf_hbm.at[pl.ds(rbase, RPC)], idx_v.at[slot])

      @pl.loop(0, RPC // 16)
      def _remap(t):
        v = idx_v[slot, pl.ds(t * 16, 16)]
        ge = (v >= HALF).astype(jnp.int32)
        idx_v[slot, pl.ds(t * 16, 16)] = 2 * v - ge * (2 * HALF - 1)

      for j in range(RPC // 80):
        pltpu.async_copy(
            table_hbm.at[idx_v.at[slot].at[pl.ds(j * 80, 80)]],
            rows_v.at[slot].at[pl.ds(j * 80, 80)],
            sems[slot],
        )

    def drain(slot):
      pltpu.make_async_copy(
          table_hbm.at[pl.ds(0, RPC)], rows_v.at[slot], sems[slot]
      ).wait()

    def pool_store(cc, slot):
      @pl.loop(0, CH)
      def _sample(s):
        base = s * H

        def body(l, accs):
          r = base + l * 5
          out = accs
          for u in range(5):
            out = tuple(
                out[v] + rows_v[slot, r + u, pl.ds(v * 16, 16)]
                for v in range(4)
            )
          return out

        accs = lax.fori_loop(
            0, H // 5, body,
            tuple(jnp.zeros((16,), jnp.float32) for _ in range(4)),
        )
        for v in range(4):
          pool_v[s, pl.ds(v * 16, 16)] = accs[v]

      sbase = wid * SPW + cc * CH
      pltpu.sync_copy(pool_v, out_hbm.at[pl.ds(sbase, CH)])

    load(0, 0)

    @pl.loop(0, NCHUNK, step=2)
    def _chunk(c):
      for b in range(2):
        cc = c + b

        @pl.when(cc + 1 < NCHUNK)
        def _():
          load(cc + 1, (b + 1) % 2)

        drain(b)
        pool_store(cc, b)

  return k(table, xf)


def _tc_transpose(tt):
  bn = TBN
  nb = HALF // bn
  nlast = (VOCAB - 1) // bn

  def body(a_ref, b_ref, o_ref):
    i128 = jnp.eye(128, dtype=jnp.float32)
    dn = (((0,), (0,)), ((), ()))
    stacked = jnp.concatenate([a_ref[...], b_ref[...]], axis=0)
    o_ref[...] = lax.dot_general(
        stacked, i128, dn, preferred_element_type=jnp.float32
    )

  return pl.pallas_call(
      body,
      grid=(nb,),
      in_specs=[
          pl.BlockSpec((D, bn), lambda i: (0, i)),
          pl.BlockSpec((D, bn), lambda i: (0, jnp.minimum(i + nb, nlast))),
      ],
      out_specs=pl.BlockSpec((bn, 128), lambda i: (i, 0)),
      out_shape=jax.ShapeDtypeStruct((HALF, 128), jnp.float32),
  )(tt, tt)


def _tc_linear(pooled, wt_pad, b_pad):
  bm = 2048

  def body(p_ref, wt_ref, b_ref, o_ref):
    acc = jnp.dot(p_ref[...], wt_ref[...], preferred_element_type=jnp.float32)
    o_ref[...] = acc * (1.0 / H) + b_ref[...]

  return pl.pallas_call(
      body,
      grid=(B // bm,),
      in_specs=[
          pl.BlockSpec((bm, D), lambda i: (i, 0)),
          pl.BlockSpec((D, 128), lambda i: (0, 0)),
          pl.BlockSpec((1, 128), lambda i: (0, 0)),
      ],
      out_specs=pl.BlockSpec((bm, 128), lambda i: (i, 0)),
      out_shape=jax.ShapeDtypeStruct((B, 128), jnp.float32),
  )(pooled, wt_pad, b_pad)


def kernel(x, table, W, b):
  xi = x.astype(jnp.int32)
  tab = _tc_transpose(table.T).reshape(2 * HALF, D)
  xf = xi.reshape(-1)
  pooled = _sc_pool_sums(tab, xf)
  wt_pad = jnp.zeros((D, 128), jnp.float32).at[:, :NCLS].set(W.T)
  b_pad = jnp.zeros((1, 128), jnp.float32).at[:, :NCLS].set(b.reshape(1, -1))
  out = _tc_linear(pooled, wt_pad, b_pad)
  return out[:, :NCLS]


# speedup vs baseline: 2.6807x; 1.0159x over previous
import functools

import jax
import jax.numpy as jnp
from jax import lax
from jax.experimental import pallas as pl
from jax.experimental.pallas import tpu as pltpu
from jax.experimental.pallas import tpu_sc as plsc

VOCAB = 1000000
D = 64
NCLS = 100
B = 16384
H = 50

TBN = 16384
HALF = 507904

NC, NS = 2, 16
NW = NC * NS
SPW = B // NW
CH = 16
NCHUNK = SPW // CH
RPC = CH * H


def _sc_pool_sums(table, xf):
  mesh = plsc.VectorSubcoreMesh(
      core_axis_name="c", subcore_axis_name="s", num_cores=NC, num_subcores=NS
  )

  @functools.partial(
      pl.kernel,
      out_type=jax.ShapeDtypeStruct((B, D), jnp.float32),
      mesh=mesh,
      scratch_types=[
          pltpu.VMEM((2, RPC), jnp.int32),
          pltpu.VMEM((2, RPC, D), jnp.float32),
          pltpu.VMEM((CH, D), jnp.float32),
          pltpu.SemaphoreType.DMA,
          pltpu.SemaphoreType.DMA,
      ],
      compiler_params=pltpu.CompilerParams(
          use_tc_tiling_on_sc=False, needs_layout_passes=False
      ),
  )
  def k(table_hbm, xf_hbm, out_hbm, idx_v, rows_v, pool_v, sem0, sem1):
    wid = lax.axis_index("s") * NC + lax.axis_index("c")
    sems = (sem0, sem1)

    def load(cc, slot):
      rbase = (wid * SPW + cc * CH) * H
      pltpu.sync_copy(xf_hbm.at[pl.ds(rbase, RPC)], idx_v.at[slot])

      @pl.loop(0, RPC // 16)
      def _remap(t):
        v = idx_v[slot, pl.ds(t * 16, 16)]
        ge = (v >= HALF).astype(jnp.int32)
        idx_v[slot, pl.ds(t * 16, 16)] = 2 * v - ge * (2 * HALF - 1)

      for j in range(RPC // 80):
        pltpu.async_copy(
            table_hbm.at[idx_v.at[slot].at[pl.ds(j * 80, 80)]],
            rows_v.at[slot].at[pl.ds(j * 80, 80)],
            sems[slot],
        )

    def drain(slot):
      pltpu.make_async_copy(
          table_hbm.at[pl.ds(0, RPC)], rows_v.at[slot], sems[slot]
      ).wait()

    def pool_store(cc, slot):
      @pl.loop(0, CH)
      def _sample(s):
        base = s * H

        def body(l, accs):
          r = base + l * 5
          out = accs
          for u in range(5):
            out = tuple(
                out[v] + rows_v[slot, r + u, pl.ds(v * 16, 16)]
                for v in range(4)
            )
          return out

        accs = lax.fori_loop(
            0, H // 5, body,
            tuple(jnp.zeros((16,), jnp.float32) for _ in range(4)),
        )
        for v in range(4):
          pool_v[s, pl.ds(v * 16, 16)] = accs[v]

      sbase = wid * SPW + cc * CH
      pltpu.sync_copy(pool_v, out_hbm.at[pl.ds(sbase, CH)])

    load(0, 0)

    @pl.loop(0, NCHUNK, step=2)
    def _chunk(c):
      for b in range(2):
        cc = c + b

        @pl.when(cc + 1 < NCHUNK)
        def _():
          load(cc + 1, (b + 1) % 2)

        drain(b)
        pool_store(cc, b)

  return k(table, xf)


def _tc_transpose(tt):
  bn = TBN
  nb = HALF // bn
  nlast = (VOCAB - 1) // bn

  def body(a_ref, b_ref, o_ref):
    i128 = jnp.eye(128, dtype=jnp.float32)
    dn = (((0,), (0,)), ((), ()))
    stacked = jnp.concatenate([a_ref[...], b_ref[...]], axis=0)
    o_ref[...] = lax.dot_general(
        stacked, i128, dn, preferred_element_type=jnp.float32
    )

  return pl.pallas_call(
      body,
      grid=(nb,),
      in_specs=[
          pl.BlockSpec((D, bn), lambda i: (0, i)),
          pl.BlockSpec((D, bn), lambda i: (0, jnp.minimum(i + nb, nlast))),
      ],
      out_specs=pl.BlockSpec((bn, 128), lambda i: (i, 0)),
      out_shape=jax.ShapeDtypeStruct((HALF, 128), jnp.float32),
      compiler_params=pltpu.CompilerParams(vmem_limit_bytes=100 * 1024 * 1024),
  )(tt, tt)


def _tc_linear(pooled, wt_pad, b_pad):
  bm = 2048

  def body(p_ref, wt_ref, b_ref, o_ref):
    acc = jnp.dot(p_ref[...], wt_ref[...], preferred_element_type=jnp.float32)
    o_ref[...] = acc * (1.0 / H) + b_ref[...]

  return pl.pallas_call(
      body,
      grid=(B // bm,),
      in_specs=[
          pl.BlockSpec((bm, D), lambda i: (i, 0)),
          pl.BlockSpec((D, 128), lambda i: (0, 0)),
          pl.BlockSpec((1, 128), lambda i: (0, 0)),
      ],
      out_specs=pl.BlockSpec((bm, 128), lambda i: (i, 0)),
      out_shape=jax.ShapeDtypeStruct((B, 128), jnp.float32),
  )(pooled, wt_pad, b_pad)


def kernel(x, table, W, b):
  xi = x.astype(jnp.int32)
  tab = _tc_transpose(table.T).reshape(2 * HALF, D)
  xf = xi.reshape(-1)
  pooled = _sc_pool_sums(tab, xf)
  wt_pad = jnp.zeros((D, 128), jnp.float32).at[:, :NCLS].set(W.T)
  b_pad = jnp.zeros((1, 128), jnp.float32).at[:, :NCLS].set(b.reshape(1, -1))
  out = _tc_linear(pooled, wt_pad, b_pad)
  return out[:, :NCLS]


# speedup vs baseline: 2.6861x; 1.0020x over previous
import functools

import jax
import jax.numpy as jnp
from jax import lax
from jax.experimental import pallas as pl
from jax.experimental.pallas import tpu as pltpu
from jax.experimental.pallas import tpu_sc as plsc

VOCAB = 1000000
D = 64
NCLS = 100
B = 16384
H = 50

TBN = 16384
HALF = 507904

NC, NS = 2, 16
NW = NC * NS
SPW = B // NW
CH = 16
NCHUNK = SPW // CH
RPC = CH * H


def _sc_pool_sums(table, xf):
  mesh = plsc.VectorSubcoreMesh(
      core_axis_name="c", subcore_axis_name="s", num_cores=NC, num_subcores=NS
  )

  @functools.partial(
      pl.kernel,
      out_type=jax.ShapeDtypeStruct((B, D), jnp.float32),
      mesh=mesh,
      scratch_types=[
          pltpu.VMEM((2, RPC), jnp.int32),
          pltpu.VMEM((2, RPC, D), jnp.float32),
          pltpu.VMEM((CH, D), jnp.float32),
          pltpu.SemaphoreType.DMA,
          pltpu.SemaphoreType.DMA,
      ],
      compiler_params=pltpu.CompilerParams(
          use_tc_tiling_on_sc=False, needs_layout_passes=False
      ),
  )
  def k(table_hbm, xf_hbm, out_hbm, idx_v, rows_v, pool_v, sem0, sem1):
    wid = lax.axis_index("s") * NC + lax.axis_index("c")
    sems = (sem0, sem1)

    def load(cc, slot):
      rbase = (wid * SPW + cc * CH) * H
      pltpu.sync_copy(xf_hbm.at[pl.ds(rbase, RPC)], idx_v.at[slot])

      @pl.loop(0, RPC // 16)
      def _remap(t):
        v = idx_v[slot, pl.ds(t * 16, 16)]
        ge = (v >= HALF).astype(jnp.int32)
        idx_v[slot, pl.ds(t * 16, 16)] = 2 * v - ge * (2 * HALF - 1)

      for j in range(RPC // 80):
        pltpu.async_copy(
            table_hbm.at[idx_v.at[slot].at[pl.ds(j * 80, 80)]],
            rows_v.at[slot].at[pl.ds(j * 80, 80)],
            sems[slot],
        )

    def drain(slot):
      pltpu.make_async_copy(
          table_hbm.at[pl.ds(0, RPC)], rows_v.at[slot], sems[slot]
      ).wait()

    def pool_store(cc, slot):
      @pl.loop(0, CH)
      def _sample(s):
        base = s * H

        def body(l, accs):
          r = base + l * 5
          out = accs
          for u in range(5):
            out = tuple(
                out[v] + rows_v[slot, r + u, pl.ds(v * 16, 16)]
                for v in range(4)
            )
          return out

        accs = lax.fori_loop(
            0, H // 5, body,
            tuple(jnp.zeros((16,), jnp.float32) for _ in range(4)),
        )
        for v in range(4):
          pool_v[s, pl.ds(v * 16, 16)] = accs[v]

      sbase = wid * SPW + cc * CH
      pltpu.sync_copy(pool_v, out_hbm.at[pl.ds(sbase, CH)])

    load(0, 0)

    @pl.loop(0, NCHUNK, step=2)
    def _chunk(c):
      for b in range(2):
        cc = c + b

        @pl.when(cc + 1 < NCHUNK)
        def _():
          load(cc + 1, (b + 1) % 2)

        drain(b)
        pool_store(cc, b)

  return k(table, xf)


def _tc_transpose(tt):
  bn = TBN
  nb = HALF // bn
  nlast = (VOCAB - 1) // bn

  def body(a_ref, b_ref, o_ref):
    i128 = jnp.eye(128, dtype=jnp.bfloat16)
    dn = (((0,), (0,)), ((), ()))
    stacked = jnp.concatenate([a_ref[...], b_ref[...]], axis=0)
    o_ref[...] = lax.dot_general(
        stacked.astype(jnp.bfloat16), i128, dn,
        preferred_element_type=jnp.float32,
    )

  return pl.pallas_call(
      body,
      grid=(nb,),
      in_specs=[
          pl.BlockSpec((D, bn), lambda i: (0, i)),
          pl.BlockSpec((D, bn), lambda i: (0, jnp.minimum(i + nb, nlast))),
      ],
      out_specs=pl.BlockSpec((bn, 128), lambda i: (i, 0)),
      out_shape=jax.ShapeDtypeStruct((HALF, 128), jnp.float32),
      compiler_params=pltpu.CompilerParams(vmem_limit_bytes=100 * 1024 * 1024),
  )(tt, tt)


def _tc_linear(pooled, wt_pad, b_pad):
  bm = 2048

  def body(p_ref, wt_ref, b_ref, o_ref):
    acc = jnp.dot(p_ref[...], wt_ref[...], preferred_element_type=jnp.float32)
    o_ref[...] = acc * (1.0 / H) + b_ref[...]

  return pl.pallas_call(
      body,
      grid=(B // bm,),
      in_specs=[
          pl.BlockSpec((bm, D), lambda i: (i, 0)),
          pl.BlockSpec((D, 128), lambda i: (0, 0)),
          pl.BlockSpec((1, 128), lambda i: (0, 0)),
      ],
      out_specs=pl.BlockSpec((bm, 128), lambda i: (i, 0)),
      out_shape=jax.ShapeDtypeStruct((B, 128), jnp.float32),
  )(pooled, wt_pad, b_pad)


def kernel(x, table, W, b):
  xi = x.astype(jnp.int32)
  tab = _tc_transpose(table.T).reshape(2 * HALF, D)
  xf = xi.reshape(-1)
  pooled = _sc_pool_sums(tab, xf)
  wt_pad = jnp.zeros((D, 128), jnp.float32).at[:, :NCLS].set(W.T)
  b_pad = jnp.zeros((1, 128), jnp.float32).at[:, :NCLS].set(b.reshape(1, -1))
  out = _tc_linear(pooled, wt_pad, b_pad)
  return out[:, :NCLS]


# speedup vs baseline: 3.0980x; 1.1533x over previous
import functools

import jax
import jax.numpy as jnp
from jax import lax
from jax.experimental import pallas as pl
from jax.experimental.pallas import tpu as pltpu
from jax.experimental.pallas import tpu_sc as plsc

VOCAB = 1000000
D = 64
NCLS = 100
B = 16384
H = 50

TBN = 8192
NB = 31
Q = NB * TBN
TOT = 4 * Q

NC, NS = 2, 16
NW = NC * NS
SPW = B // NW
CH = 16
NCHUNK = SPW // CH
RPC = CH * H


def _sc_pool_sums(table, xf):
  mesh = plsc.VectorSubcoreMesh(
      core_axis_name="c", subcore_axis_name="s", num_cores=NC, num_subcores=NS
  )
  fmt = plsc.PackFormat.INTERLEAVED

  @functools.partial(
      pl.kernel,
      out_type=jax.ShapeDtypeStruct((B, D), jnp.float32),
      mesh=mesh,
      scratch_types=[
          pltpu.VMEM((2, RPC), jnp.int32),
          pltpu.VMEM((2, RPC, 32), jnp.int32),
          pltpu.VMEM((CH, D), jnp.float32),
          pltpu.SemaphoreType.DMA,
          pltpu.SemaphoreType.DMA,
      ],
      compiler_params=pltpu.CompilerParams(
          use_tc_tiling_on_sc=False, needs_layout_passes=False
      ),
  )
  def k(table_hbm, xf_hbm, out_hbm, idx_v, rows_v, pool_v, sem0, sem1):
    wid = lax.axis_index("s") * NC + lax.axis_index("c")
    sems = (sem0, sem1)

    def load(cc, slot):
      rbase = (wid * SPW + cc * CH) * H
      pltpu.sync_copy(xf_hbm.at[pl.ds(rbase, RPC)], idx_v.at[slot])

      @pl.loop(0, RPC // 16)
      def _remap(t):
        v = idx_v[slot, pl.ds(t * 16, 16)]
        kq = (
            (v >= Q).astype(jnp.int32)
            + (v >= 2 * Q).astype(jnp.int32)
            + (v >= 3 * Q).astype(jnp.int32)
        )
        idx_v[slot, pl.ds(t * 16, 16)] = 4 * v - kq * (4 * Q - 1)

      for j in range(RPC // 80):
        pltpu.async_copy(
            table_hbm.at[idx_v.at[slot].at[pl.ds(j * 80, 80)]],
            rows_v.at[slot].at[pl.ds(j * 80, 80)],
            sems[slot],
        )

    def drain(slot):
      pltpu.make_async_copy(
          table_hbm.at[pl.ds(0, RPC)], rows_v.at[slot], sems[slot]
      ).wait()

    def pool_store(cc, slot):
      @pl.loop(0, CH)
      def _sample(s):
        base = s * H

        def body(l, accs):
          r = base + l * 5
          out = accs
          for u in range(5):
            new = []
            for v in range(2):
              bf = plsc.bitcast(
                  rows_v[slot, r + u, pl.ds(v * 16, 16)], jnp.bfloat16
              )
              ua, ub = plsc.unpack(bf, format=fmt)
              new.append(out[2 * v] + ua)
              new.append(out[2 * v + 1] + ub)
            out = tuple(new)
          return out

        accs = lax.fori_loop(
            0, H // 5, body,
            tuple(jnp.zeros((16,), jnp.float32) for _ in range(4)),
        )
        for v in range(4):
          pool_v[s, pl.ds(v * 16, 16)] = accs[v]

      sbase = wid * SPW + cc * CH
      pltpu.sync_copy(pool_v, out_hbm.at[pl.ds(sbase, CH)])

    load(0, 0)

    @pl.loop(0, NCHUNK, step=2)
    def _chunk(c):
      for b in range(2):
        cc = c + b

        @pl.when(cc + 1 < NCHUNK)
        def _():
          load(cc + 1, (b + 1) % 2)

        drain(b)
        pool_store(cc, b)

  return k(table, xf)


def _tc_repack(tt, pe, po):
  bn = TBN
  nlast = (VOCAB - 1) // bn
  dn = (((0,), (0,)), ((), ()))

  def body(a_ref, b_ref, c_ref, d_ref, pe_ref, po_ref, o_ref):
    s4 = jnp.concatenate(
        [a_ref[...], b_ref[...], c_ref[...], d_ref[...]], axis=0
    ).astype(jnp.bfloat16)
    ev = lax.dot_general(s4, pe_ref[...], dn, preferred_element_type=jnp.float32)
    od = lax.dot_general(s4, po_ref[...], dn, preferred_element_type=jnp.float32)
    ue = lax.bitcast_convert_type(ev, jnp.uint32)
    uo = lax.bitcast_convert_type(od, jnp.uint32)
    w = (ue >> jnp.uint32(16)) | (uo & jnp.uint32(0xFFFF0000))
    o_ref[...] = lax.bitcast_convert_type(w, jnp.int32)

  def make_map(kq):
    return lambda i: (0, jnp.minimum(kq * NB + i, nlast))

  return pl.pallas_call(
      body,
      grid=(NB,),
      in_specs=[
          pl.BlockSpec((D, bn), make_map(0)),
          pl.BlockSpec((D, bn), make_map(1)),
          pl.BlockSpec((D, bn), make_map(2)),
          pl.BlockSpec((D, bn), make_map(3)),
          pl.BlockSpec((256, 128), lambda i: (0, 0)),
          pl.BlockSpec((256, 128), lambda i: (0, 0)),
      ],
      out_specs=pl.BlockSpec((bn, 128), lambda i: (i, 0)),
      out_shape=jax.ShapeDtypeStruct((Q, 128), jnp.int32),
      compiler_params=pltpu.CompilerParams(vmem_limit_bytes=100 * 1024 * 1024),
  )(tt, tt, tt, tt, pe, po)


def _tc_linear(pooled, wt_perm, b_pad):
  bm = 2048

  def body(p_ref, wt_ref, b_ref, o_ref):
    acc = jnp.dot(p_ref[...], wt_ref[...], preferred_element_type=jnp.float32)
    o_ref[...] = acc * (1.0 / H) + b_ref[...]

  return pl.pallas_call(
      body,
      grid=(B // bm,),
      in_specs=[
          pl.BlockSpec((bm, D), lambda i: (i, 0)),
          pl.BlockSpec((D, 128), lambda i: (0, 0)),
          pl.BlockSpec((1, 128), lambda i: (0, 0)),
      ],
      out_specs=pl.BlockSpec((bm, 128), lambda i: (i, 0)),
      out_shape=jax.ShapeDtypeStruct((B, 128), jnp.float32),
  )(pooled, wt_perm, b_pad)


def kernel(x, table, W, b):
  xi = x.astype(jnp.int32)

  kk = lax.broadcasted_iota(jnp.int32, (256, 128), 0)
  jj = lax.broadcasted_iota(jnp.int32, (256, 128), 1)
  pe = (kk == (jj // 32) * D + 2 * (jj % 32)).astype(jnp.bfloat16)
  po = (kk == (jj // 32) * D + 2 * (jj % 32) + 1).astype(jnp.bfloat16)

  tab = _tc_repack(table.T, pe, po).reshape(TOT, 32)
  pooled = _sc_pool_sums(tab, xi.reshape(-1))

  t16 = jnp.arange(16, dtype=jnp.int32)
  grp = jnp.concatenate([2 * t16, 2 * t16 + 1])
  perm = jnp.concatenate([grp, 32 + grp])
  wt_pad = jnp.zeros((D, 128), jnp.float32).at[:, :NCLS].set(W.T)
  wt_perm = jnp.take(wt_pad, perm, axis=0)
  b_pad = jnp.zeros((1, 128), jnp.float32).at[:, :NCLS].set(b.reshape(1, -1))
  out = _tc_linear(pooled, wt_perm, b_pad)
  return out[:, :NCLS]


# speedup vs baseline: 3.1744x; 1.0247x over previous
import functools

import jax
import jax.numpy as jnp
from jax import lax
from jax.experimental import pallas as pl
from jax.experimental.pallas import tpu as pltpu
from jax.experimental.pallas import tpu_sc as plsc

VOCAB = 1000000
D = 64
NCLS = 100
B = 16384
H = 50

TBN = 16384
NB = 16
Q = NB * TBN
TOT = 4 * Q

NC, NS = 2, 16
NW = NC * NS
SPW = B // NW
CH = 16
NCHUNK = SPW // CH
RPC = CH * H


def _sc_pool_sums(table, xf):
  mesh = plsc.VectorSubcoreMesh(
      core_axis_name="c", subcore_axis_name="s", num_cores=NC, num_subcores=NS
  )
  fmt = plsc.PackFormat.INTERLEAVED

  @functools.partial(
      pl.kernel,
      out_type=jax.ShapeDtypeStruct((B, D), jnp.float32),
      mesh=mesh,
      scratch_types=[
          pltpu.VMEM((2, RPC), jnp.int32),
          pltpu.VMEM((2, RPC, 32), jnp.int32),
          pltpu.VMEM((CH, D), jnp.float32),
          pltpu.SemaphoreType.DMA,
          pltpu.SemaphoreType.DMA,
      ],
      compiler_params=pltpu.CompilerParams(
          use_tc_tiling_on_sc=False, needs_layout_passes=False
      ),
  )
  def k(table_hbm, xf_hbm, out_hbm, idx_v, rows_v, pool_v, sem0, sem1):
    wid = lax.axis_index("s") * NC + lax.axis_index("c")
    sems = (sem0, sem1)

    def load(cc, slot):
      rbase = (wid * SPW + cc * CH) * H
      pltpu.sync_copy(xf_hbm.at[pl.ds(rbase, RPC)], idx_v.at[slot])

      @pl.loop(0, RPC // 16)
      def _remap(t):
        v = idx_v[slot, pl.ds(t * 16, 16)]
        kq = (
            (v >= Q).astype(jnp.int32)
            + (v >= 2 * Q).astype(jnp.int32)
            + (v >= 3 * Q).astype(jnp.int32)
        )
        idx_v[slot, pl.ds(t * 16, 16)] = 4 * v - kq * (4 * Q - 1)

      for j in range(RPC // 80):
        pltpu.async_copy(
            table_hbm.at[idx_v.at[slot].at[pl.ds(j * 80, 80)]],
            rows_v.at[slot].at[pl.ds(j * 80, 80)],
            sems[slot],
        )

    def drain(slot):
      pltpu.make_async_copy(
          table_hbm.at[pl.ds(0, RPC)], rows_v.at[slot], sems[slot]
      ).wait()

    def pool_store(cc, slot):
      @pl.loop(0, CH)
      def _sample(s):
        base = s * H

        def body(l, accs):
          r = base + l * 5
          out = accs
          for u in range(5):
            new = []
            for v in range(2):
              bf = plsc.bitcast(
                  rows_v[slot, r + u, pl.ds(v * 16, 16)], jnp.bfloat16
              )
              ua, ub = plsc.unpack(bf, format=fmt)
              new.append(out[2 * v] + ua)
              new.append(out[2 * v + 1] + ub)
            out = tuple(new)
          return out

        accs = lax.fori_loop(
            0, H // 5, body,
            tuple(jnp.zeros((16,), jnp.float32) for _ in range(4)),
        )
        for v in range(4):
          pool_v[s, pl.ds(v * 16, 16)] = accs[v]

      sbase = wid * SPW + cc * CH
      pltpu.sync_copy(pool_v, out_hbm.at[pl.ds(sbase, CH)])

    load(0, 0)

    @pl.loop(0, NCHUNK, step=2)
    def _chunk(c):
      for b in range(2):
        cc = c + b

        @pl.when(cc + 1 < NCHUNK)
        def _():
          load(cc + 1, (b + 1) % 2)

        drain(b)
        pool_store(cc, b)

  return k(table, xf)


def _tc_repack(tt, pe, po):
  bn = TBN
  nlast = (VOCAB - 1) // bn
  dn = (((0,), (0,)), ((), ()))

  def body(a_ref, b_ref, c_ref, d_ref, pe_ref, po_ref, o_ref):
    s4 = jnp.concatenate(
        [a_ref[...], b_ref[...], c_ref[...], d_ref[...]], axis=0
    ).astype(jnp.bfloat16)
    ev = lax.dot_general(s4, pe_ref[...], dn, preferred_element_type=jnp.float32)
    od = lax.dot_general(s4, po_ref[...], dn, preferred_element_type=jnp.float32)
    ue = lax.bitcast_convert_type(ev, jnp.uint32)
    uo = lax.bitcast_convert_type(od, jnp.uint32)
    w = (ue >> jnp.uint32(16)) | (uo & jnp.uint32(0xFFFF0000))
    o_ref[...] = lax.bitcast_convert_type(w, jnp.int32)

  def make_map(kq):
    return lambda i: (0, jnp.minimum(kq * NB + i, nlast))

  return pl.pallas_call(
      body,
      grid=(NB,),
      in_specs=[
          pl.BlockSpec((D, bn), make_map(0)),
          pl.BlockSpec((D, bn), make_map(1)),
          pl.BlockSpec((D, bn), make_map(2)),
          pl.BlockSpec((D, bn), make_map(3)),
          pl.BlockSpec((256, 128), lambda i: (0, 0)),
          pl.BlockSpec((256, 128), lambda i: (0, 0)),
      ],
      out_specs=pl.BlockSpec((bn, 128), lambda i: (i, 0)),
      out_shape=jax.ShapeDtypeStruct((Q, 128), jnp.int32),
      compiler_params=pltpu.CompilerParams(vmem_limit_bytes=100 * 1024 * 1024),
  )(tt, tt, tt, tt, pe, po)


def _tc_linear(pooled, wt_perm, b_pad):
  bm = 2048

  def body(p_ref, wt_ref, b_ref, o_ref):
    acc = jnp.dot(p_ref[...], wt_ref[...], preferred_element_type=jnp.float32)
    o_ref[...] = acc * (1.0 / H) + b_ref[...]

  return pl.pallas_call(
      body,
      grid=(B // bm,),
      in_specs=[
          pl.BlockSpec((bm, D), lambda i: (i, 0)),
          pl.BlockSpec((D, 128), lambda i: (0, 0)),
          pl.BlockSpec((1, 128), lambda i: (0, 0)),
      ],
      out_specs=pl.BlockSpec((bm, 128), lambda i: (i, 0)),
      out_shape=jax.ShapeDtypeStruct((B, 128), jnp.float32),
  )(pooled, wt_perm, b_pad)


def kernel(x, table, W, b):
  xi = x.astype(jnp.int32)

  kk = lax.broadcasted_iota(jnp.int32, (256, 128), 0)
  jj = lax.broadcasted_iota(jnp.int32, (256, 128), 1)
  pe = (kk == (jj // 32) * D + 2 * (jj % 32)).astype(jnp.bfloat16)
  po = (kk == (jj // 32) * D + 2 * (jj % 32) + 1).astype(jnp.bfloat16)

  tab = _tc_repack(table.T, pe, po).reshape(TOT, 32)
  pooled = _sc_pool_sums(tab, xi.reshape(-1))

  t16 = jnp.arange(16, dtype=jnp.int32)
  grp = jnp.concatenate([2 * t16, 2 * t16 + 1])
  perm = jnp.concatenate([grp, 32 + grp])
  wt_pad = jnp.zeros((D, 128), jnp.float32).at[:, :NCLS].set(W.T)
  wt_perm = jnp.take(wt_pad, perm, axis=0)
  b_pad = jnp.zeros((1, 128), jnp.float32).at[:, :NCLS].set(b.reshape(1, -1))
  out = _tc_linear(pooled, wt_perm, b_pad)
  return out[:, :NCLS]
